# baseline (speedup 1.0000x reference)
import functools

import jax
import jax.numpy as jnp
from jax import lax
from jax.experimental import pallas as pl
from jax.experimental.pallas import tpu as pltpu
from jax.experimental.pallas import tpu_sc as plsc

GENRES = 100000
CHANNELS = 32
BATCH = 16384

NUM_CORES = 2
NUM_SUBCORES = 16
NUM_WORKERS = NUM_CORES * NUM_SUBCORES

B_PER_W = BATCH // NUM_WORKERS
CHUNK = 128
NCHUNK = B_PER_W // CHUNK


def _gather_body(idx_hbm, table_hbm, out_hbm, idx_v, rows_v, sem):
    wid = lax.axis_index("s") * NUM_CORES + lax.axis_index("c")
    pltpu.sync_copy(idx_hbm.at[pl.ds(wid * NCHUNK, NCHUNK)], idx_v)
    copies = [
        pltpu.async_copy(table_hbm.at[idx_v.at[j]], rows_v.at[j], sem)
        for j in range(NCHUNK)
    ]
    for c in copies:
        c.wait()
    pltpu.sync_copy(rows_v, out_hbm.at[pl.ds(wid * NCHUNK, NCHUNK)])


@jax.jit
def _lookup(genre, table):
    idx2d = genre.reshape(BATCH // CHUNK, CHUNK)
    mesh = plsc.VectorSubcoreMesh(core_axis_name="c", subcore_axis_name="s")
    out = pl.kernel(
        _gather_body,
        out_type=jax.ShapeDtypeStruct((BATCH // CHUNK, CHUNK, CHANNELS),
                                      jnp.float32),
        mesh=mesh,
        scratch_types=[
            pltpu.VMEM((NCHUNK, CHUNK), jnp.int32),
            pltpu.VMEM((NCHUNK, CHUNK, CHANNELS), jnp.float32),
            pltpu.SemaphoreType.DMA,
        ],
        compiler_params=pltpu.CompilerParams(use_tc_tiling_on_sc=False),
    )(idx2d, table)
    return out.reshape(BATCH, CHANNELS, 1, 1)


def kernel(genre, table):
    return _lookup(genre, table)


# speedup vs baseline: 2.2459x; 2.2459x over previous
import functools

import jax
import jax.numpy as jnp
from jax import lax
from jax.experimental import pallas as pl
from jax.experimental.pallas import tpu as pltpu
from jax.experimental.pallas import tpu_sc as plsc

GENRES = 100000
CHANNELS = 32
BATCH = 16384

NUM_CORES = 2
NUM_SUBCORES = 16

HALF = BATCH // 2


def _embed_body(genre_hbm, table_hbm, out_hbm, chan_v, idx_v, out_v):
    ch = lax.axis_index("c") * NUM_SUBCORES + lax.axis_index("s")
    pltpu.sync_copy(table_hbm.at[ch], chan_v)
    for h in range(2):
        pltpu.sync_copy(genre_hbm.at[pl.ds(h * HALF, HALF)], idx_v)

        def body(i, carry):
            g = idx_v[pl.ds(i * 16, 16)]
            row = i // 8
            col = (i % 8) * 16
            out_v[row, pl.ds(col, 16)] = plsc.load_gather(chan_v, [g])
            return carry

        lax.fori_loop(0, HALF // 16, body, 0)
        pltpu.sync_copy(
            out_v, out_hbm.at[pl.ds(ch * 128 + h * (HALF // 128), HALF // 128)]
        )


@jax.jit
def _lookup(genre, table):
    mesh = plsc.VectorSubcoreMesh(core_axis_name="c", subcore_axis_name="s")
    out = pl.kernel(
        _embed_body,
        out_type=jax.ShapeDtypeStruct((BATCH * CHANNELS // 128, 128),
                                      jnp.float32),
        mesh=mesh,
        scratch_types=[
            pltpu.VMEM((GENRES,), jnp.float32),
            pltpu.VMEM((HALF,), jnp.int32),
            pltpu.VMEM((HALF // 128, 128), jnp.float32),
        ],
        compiler_params=pltpu.CompilerParams(
            use_tc_tiling_on_sc=True, needs_layout_passes=False
        ),
    )(genre, table.T)
    return out.reshape(CHANNELS, 1, BATCH).transpose(2, 0, 1).reshape(
        BATCH, CHANNELS, 1, 1)


def kernel(genre, table):
    return _lookup(genre, table)


# speedup vs baseline: 2.6379x; 1.1745x over previous
import functools

import jax
import jax.numpy as jnp
from jax import lax
from jax.experimental import pallas as pl
from jax.experimental.pallas import tpu as pltpu
from jax.experimental.pallas import tpu_sc as plsc

GENRES = 100000
CHANNELS = 32
BATCH = 16384

NUM_CORES = 2
NUM_SUBCORES = 16

CHUNK = 4096
NCHUNK = BATCH // CHUNK
ROWS = CHUNK // 128


def _embed_body(genre_hbm, table_hbm, out_hbm, chan_v, idx_v, out_v,
                chan_sem, idx_sem, out_sem):
    ch = lax.axis_index("c") * NUM_SUBCORES + lax.axis_index("s")
    chan_cp = pltpu.async_copy(table_hbm.at[ch], chan_v, chan_sem)
    idx_cp = pltpu.async_copy(
        genre_hbm.at[pl.ds(0, ROWS), :], idx_v.at[pl.ds(0, ROWS), :], idx_sem
    )
    chan_cp.wait()
    out_cps = []
    for q in range(NCHUNK):
        if q + 1 < NCHUNK:
            next_idx_cp = pltpu.async_copy(
                genre_hbm.at[pl.ds((q + 1) * ROWS, ROWS), :],
                idx_v.at[pl.ds(((q + 1) % 2) * ROWS, ROWS), :],
                idx_sem,
            )
        idx_cp.wait()
        if q >= 2:
            out_cps[q - 2].wait()
        ibase = (q % 2) * ROWS

        def row_body(r, carry, ibase=ibase):
            for k in range(8):
                g = idx_v[ibase + r, pl.ds(k * 16, 16)]
                out_v[ibase + r, pl.ds(k * 16, 16)] = plsc.load_gather(
                    chan_v, [g]
                )
            return carry

        lax.fori_loop(0, ROWS, row_body, 0)
        out_cps.append(
            pltpu.async_copy(
                out_v.at[pl.ds(ibase, ROWS), :],
                out_hbm.at[pl.ds(ch * 128 + q * ROWS, ROWS)],
                out_sem,
            )
        )
        if q + 1 < NCHUNK:
            idx_cp = next_idx_cp
    out_cps[-2].wait()
    out_cps[-1].wait()


@jax.jit
def _lookup(genre, table):
    mesh = plsc.VectorSubcoreMesh(core_axis_name="c", subcore_axis_name="s")
    out = pl.kernel(
        _embed_body,
        out_type=jax.ShapeDtypeStruct((BATCH * CHANNELS // 128, 128),
                                      jnp.float32),
        mesh=mesh,
        scratch_types=[
            pltpu.VMEM((GENRES,), jnp.float32),
            pltpu.VMEM((2 * ROWS, 128), jnp.int32),
            pltpu.VMEM((2 * ROWS, 128), jnp.float32),
            pltpu.SemaphoreType.DMA,
            pltpu.SemaphoreType.DMA,
            pltpu.SemaphoreType.DMA,
        ],
        compiler_params=pltpu.CompilerParams(
            use_tc_tiling_on_sc=True, needs_layout_passes=False
        ),
    )(genre.reshape(BATCH // 128, 128), table.T)
    return out.reshape(CHANNELS, 1, BATCH).transpose(2, 0, 1).reshape(
        BATCH, CHANNELS, 1, 1)


def kernel(genre, table):
    return _lookup(genre, table)
